# baseline (speedup 1.0000x reference)
import jax
import jax.numpy as jnp
from jax import lax
from jax.experimental import pallas as pl
from jax.experimental.pallas import tpu as pltpu, tpu_sc as plsc

_RADIUS = 0.2
_NSAMPLE = 32
_B, _N, _NPOINT, _C = 8, 8192, 1024, 64
_NS_TOT = _NSAMPLE + 1
_FLAT = _NPOINT * _NS_TOT
_NCH = 2 * 3 + _C

_NC, _NSUB, _L = 2, 16, 16
_NW = _NC * _NSUB
_WPB = _NW // _B
_JPW = _NPOINT // _WPB
_FL2 = _JPW * _NS_TOT

_i32 = jnp.int32
_BLK = 128
_GU = 8


def _lane_iota():
    return lax.broadcasted_iota(_i32, (_L,), 0)


def _vgather(x, idx):
    return lax.gather(
        x, idx[:, None],
        lax.GatherDimensionNumbers(offset_dims=(), collapsed_slice_dims=(0,),
                                   start_index_map=(0,)),
        (1,), mode=lax.GatherScatterMode.PROMISE_IN_BOUNDS)


def _prefix_inclusive(m, lane):
    x = m
    for k in (1, 2, 4, 8):
        shifted = _vgather(x, jnp.maximum(lane - k, 0))
        x = x + jnp.where(lane >= k, shifted, 0)
    return x


def _fused_body(xyzt, newt, fps, feat, out,
                xyz_v, new_v, fps_v, idx_v, buf, tbl0, tbl1, ob0, ob1,
                si0, si1, so0, so1):
    wid = lax.axis_index("s") * _NC + lax.axis_index("c")
    b = wid // _WPB
    j0 = (wid % _WPB) * _JPW
    f0 = j0 * _NS_TOT

    for d in range(3):
        pltpu.sync_copy(xyzt.at[pl.ds((b * 3 + d) * _N, _N)],
                        xyz_v.at[pl.ds(d * _N, _N)])
        pltpu.sync_copy(newt.at[pl.ds((b * 3 + d) * _NPOINT + j0, _JPW)],
                        new_v.at[pl.ds(d * _JPW, _JPW)])
    pltpu.sync_copy(fps.at[pl.ds(b * _NPOINT + j0, _JPW)], fps_v)

    lane = _lane_iota()
    zeros = jnp.zeros((_L,), _i32)
    r2 = jnp.float32(_RADIUS * _RADIUS)

    def per_centroid(j, carry):
        jv = jnp.full((_L,), j, _i32)
        cx = plsc.load_gather(new_v, [jv])
        cy = plsc.load_gather(new_v, [jv + _JPW])
        cz = plsc.load_gather(new_v, [jv + 2 * _JPW])

        def cond(state):
            n, cntv = state
            return jnp.logical_and(jnp.all(cntv < _NSAMPLE), n < _N)

        def body(state):
            n, cntv = state
            off = cntv
            for i in range(_BLK // _L):
                base = n + i * _L
                xs = xyz_v[pl.ds(base, _L)]
                ys = xyz_v[pl.ds(_N + base, _L)]
                zs = xyz_v[pl.ds(2 * _N + base, _L)]
                dx = xs - cx
                dy = ys - cy
                dz = zs - cz
                d2 = (dx * dx + dy * dy) + dz * dz
                mask = d2 < r2
                pos = off + _prefix_inclusive(mask.astype(_i32), lane) - 1
                plsc.store_scatter(buf, [pos], lane + base, mask=mask)
                off = off + plsc.all_reduce_population_count(mask)
            return n + _BLK, off

        _, cntv = lax.while_loop(
            cond, body, (jnp.int32(0), jnp.zeros((_L,), _i32)))

        firstv = plsc.load_gather(buf, [zeros])
        firstv = jnp.where(cntv > 0, firstv, zeros)
        fpsv = plsc.load_gather(fps_v, [jv])
        base = j * _NS_TOT
        basev = jnp.full((_L,), base, _i32)
        plsc.store_scatter(idx_v, [basev], fpsv, mask=lane == 0)
        for g in range(2):
            bv = buf[pl.ds(g * _L, _L)]
            vals = jnp.where(lane + g * _L < cntv, bv, firstv)
            plsc.store_scatter(idx_v, [basev + 1 + g * _L + lane], vals)
        return carry

    lax.fori_loop(0, _JPW, per_centroid, jnp.int32(0))

    def feat_src(ch):
        return feat.at[pl.ds((b * _C + ch - 6) * _N, _N)]

    def out_dst(ch):
        return out.at[pl.ds((b * _NCH + ch) * _FLAT + f0, _FL2)]

    def gather_loop(ob, tbl_off, center, src_ref):
        def chunk(t, carry):
            k0 = t * (_L * _GU)
            for u in range(_GU):
                k = k0 + u * _L
                iv = idx_v[pl.ds(k, _L)]
                if tbl_off:
                    iv = iv + tbl_off
                vals = plsc.load_gather(src_ref, [iv])
                if center:
                    jv = lax.div(jnp.full((_L,), k, _i32) + lane,
                                 jnp.full((_L,), _NS_TOT, _i32))
                    vals = vals - plsc.load_gather(new_v, [jv + tbl_off // _N * _JPW])
                ob[pl.ds(k, _L)] = vals
            return carry

        lax.fori_loop(0, _FL2 // (_L * _GU), chunk, jnp.int32(0))

    pltpu.async_copy(feat_src(6), tbl0, si0)
    pltpu.async_copy(feat_src(7), tbl1, si1)

    for d in range(3):
        gather_loop(ob0, d * _N, True, xyz_v)
        pltpu.sync_copy(ob0, out_dst(d))
        pltpu.sync_copy(ob0, out_dst(d + 3))

    def pair(t, carry):
        ch0 = 6 + 2 * t
        for (tbl, si, ob, so, ch) in ((tbl0, si0, ob0, so0, ch0),
                                      (tbl1, si1, ob1, so1, ch0 + 1)):
            pltpu.make_async_copy(feat_src(ch), tbl, si).wait()

            @pl.when(t > 0)
            def _():
                pltpu.make_async_copy(ob, out_dst(ch - 2), so).wait()

            gather_loop(ob, 0, False, tbl)

            @pl.when(t < (_C // 2 - 1))
            def _():
                pltpu.async_copy(feat_src(ch + 2), tbl, si)

            pltpu.async_copy(ob, out_dst(ch), so)
        return carry

    lax.fori_loop(0, _C // 2, pair, jnp.int32(0))
    pltpu.make_async_copy(ob0, out_dst(_NCH - 2), so0).wait()
    pltpu.make_async_copy(ob1, out_dst(_NCH - 1), so1).wait()


@jax.jit
def kernel(xyz, new_xyz, features, fps_idx):
    xyzt = jnp.transpose(xyz, (0, 2, 1)).reshape(-1)
    newt = jnp.transpose(new_xyz, (0, 2, 1)).reshape(-1)
    feat = features.reshape(-1)
    fps = fps_idx.reshape(-1)
    mesh = plsc.VectorSubcoreMesh(core_axis_name="c", subcore_axis_name="s")

    fused = pl.kernel(
        _fused_body,
        out_type=jax.ShapeDtypeStruct((_B * _NCH * _FLAT,), jnp.float32),
        mesh=mesh,
        compiler_params=pltpu.CompilerParams(needs_layout_passes=False),
        scratch_types=[
            pltpu.VMEM((3 * _N,), jnp.float32),
            pltpu.VMEM((3 * _JPW,), jnp.float32),
            pltpu.VMEM((_JPW,), _i32),
            pltpu.VMEM((_FL2,), _i32),
            pltpu.VMEM((_NSAMPLE + _BLK,), _i32),
            pltpu.VMEM((_N,), jnp.float32),
            pltpu.VMEM((_N,), jnp.float32),
            pltpu.VMEM((_FL2,), jnp.float32),
            pltpu.VMEM((_FL2,), jnp.float32),
            pltpu.SemaphoreType.DMA,
            pltpu.SemaphoreType.DMA,
            pltpu.SemaphoreType.DMA,
            pltpu.SemaphoreType.DMA,
        ],
    )
    out = fused(xyzt, newt, fps, feat)
    return out.reshape(_B, _NCH, _NPOINT, _NS_TOT)


# speedup vs baseline: 1.0961x; 1.0961x over previous
import jax
import jax.numpy as jnp
from jax import lax
from jax.experimental import pallas as pl
from jax.experimental.pallas import tpu as pltpu, tpu_sc as plsc

_RADIUS = 0.2
_NSAMPLE = 32
_B, _N, _NPOINT, _C = 8, 8192, 1024, 64
_NS_TOT = _NSAMPLE + 1
_FLAT = _NPOINT * _NS_TOT
_NCH = 2 * 3 + _C

_NC, _NSUB, _L = 2, 16, 16
_NW = _NC * _NSUB
_WPB = _NW // _B
_JPW = _NPOINT // _WPB
_FL2 = _JPW * _NS_TOT

_i32 = jnp.int32
_BLK = 256
_GU = 8


def _lane_iota():
    return lax.broadcasted_iota(_i32, (_L,), 0)


def _fused_body(xyzt, newt, fps, feat, out,
                xyz_v, new_v, fps_v, idx_v, buf, tbl0, tbl1, ob0, ob1,
                si0, si1, so0, so1):
    wid = lax.axis_index("s") * _NC + lax.axis_index("c")
    b = wid // _WPB
    j0 = (wid % _WPB) * _JPW
    f0 = j0 * _NS_TOT

    for d in range(3):
        pltpu.sync_copy(xyzt.at[pl.ds((b * 3 + d) * _N, _N)],
                        xyz_v.at[pl.ds(d * _N, _N)])
        pltpu.sync_copy(newt.at[pl.ds((b * 3 + d) * _NPOINT + j0, _JPW)],
                        new_v.at[pl.ds(d * _JPW, _JPW)])
    pltpu.sync_copy(fps.at[pl.ds(b * _NPOINT + j0, _JPW)], fps_v)

    lane = _lane_iota()
    zeros = jnp.zeros((_L,), _i32)
    r2 = jnp.float32(_RADIUS * _RADIUS)

    def per_centroid(j, carry):
        jv = jnp.full((_L,), j, _i32)
        cx = plsc.load_gather(new_v, [jv])
        cy = plsc.load_gather(new_v, [jv + _JPW])
        cz = plsc.load_gather(new_v, [jv + 2 * _JPW])

        def cond(state):
            n, cntv = state
            return jnp.logical_and(jnp.max(cntv) < _NSAMPLE, n < _N)

        def body(state):
            n, cntv = state
            off = cntv
            for i in range(_BLK // _L):
                base = n + i * _L
                xs = xyz_v[pl.ds(base, _L)]
                ys = xyz_v[pl.ds(_N + base, _L)]
                zs = xyz_v[pl.ds(2 * _N + base, _L)]
                dx = xs - cx
                dy = ys - cy
                dz = zs - cz
                d2 = (dx * dx + dy * dy) + dz * dz
                mask = d2 < r2
                pos = off + plsc.cumsum(mask.astype(_i32)) - 1
                plsc.store_scatter(buf, [pos], lane + base, mask=mask)
                off = off + plsc.all_reduce_population_count(mask)
            return n + _BLK, off

        _, cntv = lax.while_loop(
            cond, body, (jnp.int32(0), jnp.zeros((_L,), _i32)))

        firstv = plsc.load_gather(buf, [zeros])
        firstv = jnp.where(cntv > 0, firstv, zeros)
        fpsv = plsc.load_gather(fps_v, [jv])
        base = j * _NS_TOT
        basev = jnp.full((_L,), base, _i32)
        plsc.store_scatter(idx_v, [basev], fpsv, mask=lane == 0)
        for g in range(2):
            bv = buf[pl.ds(g * _L, _L)]
            vals = jnp.where(lane + g * _L < cntv, bv, firstv)
            plsc.store_scatter(idx_v, [basev + 1 + g * _L + lane], vals)
        return carry

    lax.fori_loop(0, _JPW, per_centroid, jnp.int32(0))

    def feat_src(ch):
        return feat.at[pl.ds((b * _C + ch - 6) * _N, _N)]

    def out_dst(ch):
        return out.at[pl.ds((b * _NCH + ch) * _FLAT + f0, _FL2)]

    def gather_loop(ob, tbl_off, center, src_ref):
        def chunk(t, carry):
            k0 = t * (_L * _GU)
            for u in range(_GU):
                k = k0 + u * _L
                iv = idx_v[pl.ds(k, _L)]
                if tbl_off:
                    iv = iv + tbl_off
                vals = plsc.load_gather(src_ref, [iv])
                if center:
                    jv = lax.div(jnp.full((_L,), k, _i32) + lane,
                                 jnp.full((_L,), _NS_TOT, _i32))
                    vals = vals - plsc.load_gather(new_v, [jv + tbl_off // _N * _JPW])
                ob[pl.ds(k, _L)] = vals
            return carry

        lax.fori_loop(0, _FL2 // (_L * _GU), chunk, jnp.int32(0))

    pltpu.async_copy(feat_src(6), tbl0, si0)
    pltpu.async_copy(feat_src(7), tbl1, si1)

    for d in range(3):
        gather_loop(ob0, d * _N, True, xyz_v)
        pltpu.sync_copy(ob0, out_dst(d))
        pltpu.sync_copy(ob0, out_dst(d + 3))

    def pair(t, carry):
        ch0 = 6 + 2 * t
        for (tbl, si, ob, so, ch) in ((tbl0, si0, ob0, so0, ch0),
                                      (tbl1, si1, ob1, so1, ch0 + 1)):
            pltpu.make_async_copy(feat_src(ch), tbl, si).wait()

            @pl.when(t > 0)
            def _():
                pltpu.make_async_copy(ob, out_dst(ch - 2), so).wait()

            gather_loop(ob, 0, False, tbl)

            @pl.when(t < (_C // 2 - 1))
            def _():
                pltpu.async_copy(feat_src(ch + 2), tbl, si)

            pltpu.async_copy(ob, out_dst(ch), so)
        return carry

    lax.fori_loop(0, _C // 2, pair, jnp.int32(0))
    pltpu.make_async_copy(ob0, out_dst(_NCH - 2), so0).wait()
    pltpu.make_async_copy(ob1, out_dst(_NCH - 1), so1).wait()


@jax.jit
def kernel(xyz, new_xyz, features, fps_idx):
    xyzt = jnp.transpose(xyz, (0, 2, 1)).reshape(-1)
    newt = jnp.transpose(new_xyz, (0, 2, 1)).reshape(-1)
    feat = features.reshape(-1)
    fps = fps_idx.reshape(-1)
    mesh = plsc.VectorSubcoreMesh(core_axis_name="c", subcore_axis_name="s")

    fused = pl.kernel(
        _fused_body,
        out_type=jax.ShapeDtypeStruct((_B * _NCH * _FLAT,), jnp.float32),
        mesh=mesh,
        compiler_params=pltpu.CompilerParams(needs_layout_passes=False),
        scratch_types=[
            pltpu.VMEM((3 * _N,), jnp.float32),
            pltpu.VMEM((3 * _JPW,), jnp.float32),
            pltpu.VMEM((_JPW,), _i32),
            pltpu.VMEM((_FL2,), _i32),
            pltpu.VMEM((_NSAMPLE + _BLK,), _i32),
            pltpu.VMEM((_N,), jnp.float32),
            pltpu.VMEM((_N,), jnp.float32),
            pltpu.VMEM((_FL2,), jnp.float32),
            pltpu.VMEM((_FL2,), jnp.float32),
            pltpu.SemaphoreType.DMA,
            pltpu.SemaphoreType.DMA,
            pltpu.SemaphoreType.DMA,
            pltpu.SemaphoreType.DMA,
        ],
    )
    out = fused(xyzt, newt, fps, feat)
    return out.reshape(_B, _NCH, _NPOINT, _NS_TOT)


# speedup vs baseline: 1.7322x; 1.5803x over previous
import jax
import jax.numpy as jnp
from jax import lax
from jax.experimental import pallas as pl
from jax.experimental.pallas import tpu as pltpu, tpu_sc as plsc

_RADIUS = 0.2
_NSAMPLE = 32
_B, _N, _NPOINT, _C = 8, 8192, 1024, 64
_NS_TOT = _NSAMPLE + 1
_FLAT = _NPOINT * _NS_TOT
_NCH = 2 * 3 + _C

_NC, _NSUB, _L = 2, 16, 16
_NW = _NC * _NSUB
_WPB = _NW // _B
_JPW = _NPOINT // _WPB
_FL2 = _JPW * _NS_TOT

_i32 = jnp.int32
_BLK = 256
_GU = 8


def _lane_iota():
    return lax.broadcasted_iota(_i32, (_L,), 0)


def _fused_body(xyzt, newt, fps, feat, out,
                xyz_v, new_v, fps_v, idx_v, buf, tbl0, tbl1, ob0, ob1,
                si0, si1, so0, so1):
    wid = lax.axis_index("s") * _NC + lax.axis_index("c")
    b = wid // _WPB
    j0 = (wid % _WPB) * _JPW
    f0 = j0 * _NS_TOT

    for d in range(3):
        pltpu.sync_copy(xyzt.at[pl.ds((b * 3 + d) * _N, _N)],
                        xyz_v.at[pl.ds(d * _N, _N)])
        pltpu.sync_copy(newt.at[pl.ds((b * 3 + d) * _NPOINT + j0, _JPW)],
                        new_v.at[pl.ds(d * _JPW, _JPW)])
    pltpu.sync_copy(fps.at[pl.ds(b * _NPOINT + j0, _JPW)], fps_v)

    lane = _lane_iota()
    zeros = jnp.zeros((_L,), _i32)
    r2 = jnp.float32(_RADIUS * _RADIUS)

    def per_centroid(j, carry):
        jv = jnp.full((_L,), j, _i32)
        cx = plsc.load_gather(new_v, [jv])
        cy = plsc.load_gather(new_v, [jv + _JPW])
        cz = plsc.load_gather(new_v, [jv + 2 * _JPW])

        def cond(state):
            n, cntv = state
            return jnp.logical_and(jnp.max(cntv) < _NSAMPLE, n < _N)

        def body(state):
            n, cntv = state

            def chunk(rel, off):
                base = n + rel
                xs = xyz_v[pl.ds(base, _L)]
                ys = xyz_v[pl.ds(_N + base, _L)]
                zs = xyz_v[pl.ds(2 * _N + base, _L)]
                dx = xs - cx
                dy = ys - cy
                dz = zs - cz
                d2 = (dx * dx + dy * dy) + dz * dz
                mask = d2 < r2
                pos = off + plsc.cumsum(mask.astype(_i32)) - 1
                plsc.store_scatter(buf, [pos], lane + base, mask=mask)
                return off + plsc.all_reduce_population_count(mask)

            off = plsc.parallel_loop(0, _BLK, step=_L, unroll=8,
                                     carry=cntv)(chunk)
            return n + _BLK, off

        _, cntv = lax.while_loop(
            cond, body, (jnp.int32(0), jnp.zeros((_L,), _i32)))

        firstv = plsc.load_gather(buf, [zeros])
        firstv = jnp.where(cntv > 0, firstv, zeros)
        fpsv = plsc.load_gather(fps_v, [jv])
        base = j * _NS_TOT
        basev = jnp.full((_L,), base, _i32)
        plsc.store_scatter(idx_v, [basev], fpsv, mask=lane == 0)
        for g in range(2):
            bv = buf[pl.ds(g * _L, _L)]
            vals = jnp.where(lane + g * _L < cntv, bv, firstv)
            plsc.store_scatter(idx_v, [basev + 1 + g * _L + lane], vals)
        return carry

    lax.fori_loop(0, _JPW, per_centroid, jnp.int32(0))

    def feat_src(ch):
        return feat.at[pl.ds((b * _C + ch - 6) * _N, _N)]

    def out_dst(ch):
        return out.at[pl.ds((b * _NCH + ch) * _FLAT + f0, _FL2)]

    def gather_loop(ob, tbl_off, center, src_ref):
        def chunk(k):
            iv = idx_v[pl.ds(k, _L)]
            if tbl_off:
                iv = iv + tbl_off
            vals = plsc.load_gather(src_ref, [iv])
            if center:
                jv = lax.div(jnp.full((_L,), k, _i32) + lane,
                             jnp.full((_L,), _NS_TOT, _i32))
                vals = vals - plsc.load_gather(new_v, [jv + tbl_off // _N * _JPW])
            ob[pl.ds(k, _L)] = vals

        plsc.parallel_loop(0, _FL2, step=_L, unroll=_GU)(chunk)

    pltpu.async_copy(feat_src(6), tbl0, si0)
    pltpu.async_copy(feat_src(7), tbl1, si1)

    for d in range(3):
        gather_loop(ob0, d * _N, True, xyz_v)
        pltpu.sync_copy(ob0, out_dst(d))
        pltpu.sync_copy(ob0, out_dst(d + 3))

    def pair(t, carry):
        ch0 = 6 + 2 * t
        for (tbl, si, ob, so, ch) in ((tbl0, si0, ob0, so0, ch0),
                                      (tbl1, si1, ob1, so1, ch0 + 1)):
            pltpu.make_async_copy(feat_src(ch), tbl, si).wait()

            @pl.when(t > 0)
            def _():
                pltpu.make_async_copy(ob, out_dst(ch - 2), so).wait()

            gather_loop(ob, 0, False, tbl)

            @pl.when(t < (_C // 2 - 1))
            def _():
                pltpu.async_copy(feat_src(ch + 2), tbl, si)

            pltpu.async_copy(ob, out_dst(ch), so)
        return carry

    lax.fori_loop(0, _C // 2, pair, jnp.int32(0))
    pltpu.make_async_copy(ob0, out_dst(_NCH - 2), so0).wait()
    pltpu.make_async_copy(ob1, out_dst(_NCH - 1), so1).wait()


@jax.jit
def kernel(xyz, new_xyz, features, fps_idx):
    xyzt = jnp.transpose(xyz, (0, 2, 1)).reshape(-1)
    newt = jnp.transpose(new_xyz, (0, 2, 1)).reshape(-1)
    feat = features.reshape(-1)
    fps = fps_idx.reshape(-1)
    mesh = plsc.VectorSubcoreMesh(core_axis_name="c", subcore_axis_name="s")

    fused = pl.kernel(
        _fused_body,
        out_type=jax.ShapeDtypeStruct((_B * _NCH * _FLAT,), jnp.float32),
        mesh=mesh,
        compiler_params=pltpu.CompilerParams(needs_layout_passes=False),
        scratch_types=[
            pltpu.VMEM((3 * _N,), jnp.float32),
            pltpu.VMEM((3 * _JPW,), jnp.float32),
            pltpu.VMEM((_JPW,), _i32),
            pltpu.VMEM((_FL2,), _i32),
            pltpu.VMEM((_NSAMPLE + _BLK,), _i32),
            pltpu.VMEM((_N,), jnp.float32),
            pltpu.VMEM((_N,), jnp.float32),
            pltpu.VMEM((_FL2,), jnp.float32),
            pltpu.VMEM((_FL2,), jnp.float32),
            pltpu.SemaphoreType.DMA,
            pltpu.SemaphoreType.DMA,
            pltpu.SemaphoreType.DMA,
            pltpu.SemaphoreType.DMA,
        ],
    )
    out = fused(xyzt, newt, fps, feat)
    return out.reshape(_B, _NCH, _NPOINT, _NS_TOT)


# speedup vs baseline: 2.4375x; 1.4071x over previous
import jax
import jax.numpy as jnp
from jax import lax
from jax.experimental import pallas as pl
from jax.experimental.pallas import tpu as pltpu, tpu_sc as plsc

_RADIUS = 0.2
_NSAMPLE = 32
_B, _N, _NPOINT, _C = 8, 8192, 1024, 64
_NS_TOT = _NSAMPLE + 1
_FLAT = _NPOINT * _NS_TOT
_NCH = 2 * 3 + _C

_NC, _NSUB, _L = 2, 16, 16
_NW = _NC * _NSUB
_WPB = _NW // _B
_JPW = _NPOINT // _WPB
_FL2 = _JPW * _NS_TOT

_i32 = jnp.int32
_BLK = 256
_GU = 8


def _lane_iota():
    return lax.broadcasted_iota(_i32, (_L,), 0)


def _fused_body(xyzt, newt, fps, feat, out,
                xyz_v, new_v, fps_v, idx_v, buf, tbl0, tbl1, ob0, ob1,
                si0, si1, so0, so1):
    wid = lax.axis_index("s") * _NC + lax.axis_index("c")
    b = wid // _WPB
    j0 = (wid % _WPB) * _JPW
    f0 = j0 * _NS_TOT

    for d in range(3):
        pltpu.sync_copy(xyzt.at[pl.ds((b * 3 + d) * _N, _N)],
                        xyz_v.at[pl.ds(d * _N, _N)])
        pltpu.sync_copy(newt.at[pl.ds((b * 3 + d) * _NPOINT + j0, _JPW)],
                        new_v.at[pl.ds(d * _JPW, _JPW)])
    pltpu.sync_copy(fps.at[pl.ds(b * _NPOINT + j0, _JPW)], fps_v)

    lane = _lane_iota()
    zeros = jnp.zeros((_L,), _i32)
    r2 = jnp.float32(_RADIUS * _RADIUS)

    def per_centroid(j, carry):
        jv = jnp.full((_L,), j, _i32)
        cx = plsc.load_gather(new_v, [jv])
        cy = plsc.load_gather(new_v, [jv + _JPW])
        cz = plsc.load_gather(new_v, [jv + 2 * _JPW])

        def cond(state):
            n, cntv = state
            return jnp.logical_and(jnp.max(cntv) < _NSAMPLE, n < _N)

        def body(state):
            n, cntv = state

            def chunk(rel, off):
                base = n + rel
                xs = xyz_v[pl.ds(base, _L)]
                ys = xyz_v[pl.ds(_N + base, _L)]
                zs = xyz_v[pl.ds(2 * _N + base, _L)]
                dx = xs - cx
                dy = ys - cy
                dz = zs - cz
                d2 = (dx * dx + dy * dy) + dz * dz
                mask = d2 < r2
                pos = off + plsc.cumsum(mask.astype(_i32)) - 1
                plsc.store_scatter(buf, [pos], lane + base, mask=mask)
                return off + plsc.all_reduce_population_count(mask)

            off = plsc.parallel_loop(0, _BLK, step=_L, unroll=8,
                                     carry=cntv)(chunk)
            return n + _BLK, off

        _, cntv = lax.while_loop(
            cond, body, (jnp.int32(0), jnp.zeros((_L,), _i32)))

        firstv = plsc.load_gather(buf, [zeros])
        firstv = jnp.where(cntv > 0, firstv, zeros)
        fpsv = plsc.load_gather(fps_v, [jv])
        base = j * _NS_TOT
        basev = jnp.full((_L,), base, _i32)
        plsc.store_scatter(idx_v, [basev], fpsv, mask=lane == 0)
        for g in range(2):
            bv = buf[pl.ds(g * _L, _L)]
            vals = jnp.where(lane + g * _L < cntv, bv, firstv)
            plsc.store_scatter(idx_v, [basev + 1 + g * _L + lane], vals)
        return carry

    lax.fori_loop(0, _JPW, per_centroid, jnp.int32(0))

    def feat_src(ch):
        return feat.at[pl.ds((b * _C + ch - 6) * _N, _N)]

    def out_dst(ch):
        return out.at[b, ch, pl.ds(j0, _JPW), :]

    def gather_loop(ob, tbl_off, center, src_ref):
        def chunk(k):
            iv = idx_v[pl.ds(k, _L)]
            if tbl_off:
                iv = iv + tbl_off
            vals = plsc.load_gather(src_ref, [iv])
            flat = jnp.full((_L,), k, _i32) + lane
            jv = lax.div(flat, jnp.full((_L,), _NS_TOT, _i32))
            if center:
                vals = vals - plsc.load_gather(new_v, [jv + tbl_off // _N * _JPW])
            plsc.store_scatter(ob, [jv, flat - jv * _NS_TOT], vals)

        plsc.parallel_loop(0, _FL2, step=_L, unroll=_GU)(chunk)

    pltpu.async_copy(feat_src(6), tbl0, si0)
    pltpu.async_copy(feat_src(7), tbl1, si1)

    for d in range(3):
        gather_loop(ob0, d * _N, True, xyz_v)
        pltpu.sync_copy(ob0, out_dst(d))
        pltpu.sync_copy(ob0, out_dst(d + 3))

    def pair(t, carry):
        ch0 = 6 + 2 * t
        for (tbl, si, ob, so, ch) in ((tbl0, si0, ob0, so0, ch0),
                                      (tbl1, si1, ob1, so1, ch0 + 1)):
            pltpu.make_async_copy(feat_src(ch), tbl, si).wait()

            @pl.when(t > 0)
            def _():
                pltpu.make_async_copy(ob, out_dst(ch - 2), so).wait()

            gather_loop(ob, 0, False, tbl)

            @pl.when(t < (_C // 2 - 1))
            def _():
                pltpu.async_copy(feat_src(ch + 2), tbl, si)

            pltpu.async_copy(ob, out_dst(ch), so)
        return carry

    lax.fori_loop(0, _C // 2, pair, jnp.int32(0))
    pltpu.make_async_copy(ob0, out_dst(_NCH - 2), so0).wait()
    pltpu.make_async_copy(ob1, out_dst(_NCH - 1), so1).wait()


@jax.jit
def kernel(xyz, new_xyz, features, fps_idx):
    xyzt = jnp.transpose(xyz, (0, 2, 1)).reshape(-1)
    newt = jnp.transpose(new_xyz, (0, 2, 1)).reshape(-1)
    feat = features.reshape(-1)
    fps = fps_idx.reshape(-1)
    mesh = plsc.VectorSubcoreMesh(core_axis_name="c", subcore_axis_name="s")

    fused = pl.kernel(
        _fused_body,
        out_type=jax.ShapeDtypeStruct((_B, _NCH, _NPOINT, _NS_TOT),
                                      jnp.float32),
        mesh=mesh,
        compiler_params=pltpu.CompilerParams(needs_layout_passes=False),
        scratch_types=[
            pltpu.VMEM((3 * _N,), jnp.float32),
            pltpu.VMEM((3 * _JPW,), jnp.float32),
            pltpu.VMEM((_JPW,), _i32),
            pltpu.VMEM((_FL2,), _i32),
            pltpu.VMEM((_NSAMPLE + _BLK,), _i32),
            pltpu.VMEM((_N,), jnp.float32),
            pltpu.VMEM((_N,), jnp.float32),
            pltpu.VMEM((_JPW, _NS_TOT), jnp.float32),
            pltpu.VMEM((_JPW, _NS_TOT), jnp.float32),
            pltpu.SemaphoreType.DMA,
            pltpu.SemaphoreType.DMA,
            pltpu.SemaphoreType.DMA,
            pltpu.SemaphoreType.DMA,
        ],
    )
    return fused(xyzt, newt, fps, feat)


# speedup vs baseline: 2.4409x; 1.0014x over previous
import jax
import jax.numpy as jnp
from jax import lax
from jax.experimental import pallas as pl
from jax.experimental.pallas import tpu as pltpu, tpu_sc as plsc

_RADIUS = 0.2
_NSAMPLE = 32
_B, _N, _NPOINT, _C = 8, 8192, 1024, 64
_NS_TOT = _NSAMPLE + 1
_FLAT = _NPOINT * _NS_TOT
_NCH = 2 * 3 + _C

_NC, _NSUB, _L = 2, 16, 16
_NW = _NC * _NSUB
_WPB = _NW // _B
_JPW = _NPOINT // _WPB
_FL2 = _JPW * _NS_TOT

_i32 = jnp.int32
_BLK = 256
_GU = 8


def _lane_iota():
    return lax.broadcasted_iota(_i32, (_L,), 0)


def _fused_body(xyzt, newt, fps, feat, out,
                xyz_v, new_v, fps_v, idx_v, buf, tbl0, tbl1, ob0, ob1,
                si0, si1, so0, so1):
    wid = lax.axis_index("s") * _NC + lax.axis_index("c")
    b = wid // _WPB
    j0 = (wid % _WPB) * _JPW
    f0 = j0 * _NS_TOT

    for d in range(3):
        pltpu.sync_copy(xyzt.at[pl.ds((b * 3 + d) * _N, _N)],
                        xyz_v.at[pl.ds(d * _N, _N)])
        pltpu.sync_copy(newt.at[pl.ds((b * 3 + d) * _NPOINT + j0, _JPW)],
                        new_v.at[pl.ds(d * _JPW, _JPW)])
    pltpu.sync_copy(fps.at[pl.ds(b * _NPOINT + j0, _JPW)], fps_v)

    lane = _lane_iota()
    zeros = jnp.zeros((_L,), _i32)
    r2 = jnp.float32(_RADIUS * _RADIUS)

    def per_centroid(j, carry):
        jv = jnp.full((_L,), j, _i32)
        cx = plsc.load_gather(new_v, [jv])
        cy = plsc.load_gather(new_v, [jv + _JPW])
        cz = plsc.load_gather(new_v, [jv + 2 * _JPW])

        def cond(state):
            n, cntv = state
            return jnp.logical_and(jnp.max(cntv) < _NSAMPLE, n < _N)

        def body(state):
            n, cntv = state

            def chunk(rel, off):
                base = n + rel
                xs = xyz_v[pl.ds(base, _L)]
                ys = xyz_v[pl.ds(_N + base, _L)]
                zs = xyz_v[pl.ds(2 * _N + base, _L)]
                dx = xs - cx
                dy = ys - cy
                dz = zs - cz
                d2 = (dx * dx + dy * dy) + dz * dz
                mask = d2 < r2
                pos = off + plsc.cumsum(mask.astype(_i32)) - 1
                plsc.store_scatter(buf, [pos], lane + base, mask=mask)
                return off + plsc.all_reduce_population_count(mask)

            off = plsc.parallel_loop(0, _BLK, step=_L, unroll=8,
                                     carry=cntv)(chunk)
            return n + _BLK, off

        _, cntv = lax.while_loop(
            cond, body, (jnp.int32(0), jnp.zeros((_L,), _i32)))

        firstv = plsc.load_gather(buf, [zeros])
        firstv = jnp.where(cntv > 0, firstv, zeros)
        fpsv = plsc.load_gather(fps_v, [jv])
        base = j * _NS_TOT
        basev = jnp.full((_L,), base, _i32)
        plsc.store_scatter(idx_v, [basev], fpsv, mask=lane == 0)
        for g in range(2):
            bv = buf[pl.ds(g * _L, _L)]
            vals = jnp.where(lane + g * _L < cntv, bv, firstv)
            plsc.store_scatter(idx_v, [basev + 1 + g * _L + lane], vals)
        return carry

    lax.fori_loop(0, _JPW, per_centroid, jnp.int32(0))

    def feat_src(ch):
        return feat.at[pl.ds((b * _C + ch - 6) * _N, _N)]

    def out_dst(ch):
        return out.at[b, ch, pl.ds(j0, _JPW), :]

    def gather_loop(ob, tbl_off, center, src_ref):
        def chunk(k):
            iv = idx_v[pl.ds(k, _L)]
            if tbl_off:
                iv = iv + tbl_off
            vals = plsc.load_gather(src_ref, [iv])
            flat = jnp.full((_L,), k, _i32) + lane
            jv = lax.div(flat, jnp.full((_L,), _NS_TOT, _i32))
            if center:
                vals = vals - plsc.load_gather(new_v, [jv + tbl_off // _N * _JPW])
            plsc.store_scatter(ob, [jv, flat - jv * _NS_TOT], vals)

        plsc.parallel_loop(0, _FL2, step=_L, unroll=_GU)(chunk)

    pltpu.async_copy(feat_src(6), tbl0, si0)
    pltpu.async_copy(feat_src(7), tbl1, si1)

    for d in range(3):
        gather_loop(ob0, d * _N, True, xyz_v)
        pltpu.sync_copy(ob0, out_dst(d))
        pltpu.sync_copy(ob0, out_dst(d + 3))

    def pair(t, carry):
        ch0 = 6 + 2 * t
        for (tbl, si, ob, so, ch) in ((tbl0, si0, ob0, so0, ch0),
                                      (tbl1, si1, ob1, so1, ch0 + 1)):
            pltpu.make_async_copy(feat_src(ch), tbl, si).wait()

            @pl.when(t > 0)
            def _():
                pltpu.make_async_copy(ob, out_dst(ch - 2), so).wait()

            gather_loop(ob, 0, False, tbl)

            @pl.when(t < (_C // 2 - 1))
            def _():
                pltpu.async_copy(feat_src(ch + 2), tbl, si)

            pltpu.async_copy(ob, out_dst(ch), so)
        return carry

    lax.fori_loop(0, _C // 2, pair, jnp.int32(0))
    pltpu.make_async_copy(ob0, out_dst(_NCH - 2), so0).wait()
    pltpu.make_async_copy(ob1, out_dst(_NCH - 1), so1).wait()


@jax.jit
def kernel(xyz, new_xyz, features, fps_idx):
    xyzt = jnp.transpose(xyz, (0, 2, 1)).reshape(-1)
    newt = jnp.transpose(new_xyz, (0, 2, 1)).reshape(-1)
    feat = features.reshape(-1)
    fps = fps_idx.reshape(-1)
    mesh = plsc.VectorSubcoreMesh(core_axis_name="c", subcore_axis_name="s")

    fused = pl.kernel(
        _fused_body,
        out_type=jax.ShapeDtypeStruct((_B, _NCH, _NPOINT, _NS_TOT),
                                      jnp.float32),
        mesh=mesh,
        compiler_params=pltpu.CompilerParams(needs_layout_passes=False,
                                             use_tc_tiling_on_sc=True),
        scratch_types=[
            pltpu.VMEM((3 * _N,), jnp.float32),
            pltpu.VMEM((3 * _JPW,), jnp.float32),
            pltpu.VMEM((_JPW,), _i32),
            pltpu.VMEM((_FL2,), _i32),
            pltpu.VMEM((_NSAMPLE + _BLK,), _i32),
            pltpu.VMEM((_N,), jnp.float32),
            pltpu.VMEM((_N,), jnp.float32),
            pltpu.VMEM((_JPW, _NS_TOT), jnp.float32),
            pltpu.VMEM((_JPW, _NS_TOT), jnp.float32),
            pltpu.SemaphoreType.DMA,
            pltpu.SemaphoreType.DMA,
            pltpu.SemaphoreType.DMA,
            pltpu.SemaphoreType.DMA,
        ],
    )
    return fused(xyzt, newt, fps, feat)


# speedup vs baseline: 2.6127x; 1.0704x over previous
import jax
import jax.numpy as jnp
from jax import lax
from jax.experimental import pallas as pl
from jax.experimental.pallas import tpu as pltpu, tpu_sc as plsc

_RADIUS = 0.2
_NSAMPLE = 32
_B, _N, _NPOINT, _C = 8, 8192, 1024, 64
_NS_TOT = _NSAMPLE + 1
_FLAT = _NPOINT * _NS_TOT
_NCH = 2 * 3 + _C

_NC, _NSUB, _L = 2, 16, 16
_NW = _NC * _NSUB
_WPB = _NW // _B
_JPW = _NPOINT // _WPB
_FL2 = _JPW * _NS_TOT

_i32 = jnp.int32
_BLK = 256
_GU = 8


def _lane_iota():
    return lax.broadcasted_iota(_i32, (_L,), 0)


def _fused_body(xyzt, newt, fps, feat, out,
                xyz_v, new_v, fps_v, idx_v, buf, tbl0, tbl1, ob0, ob1,
                si0, si1, so0, so1):
    wid = lax.axis_index("s") * _NC + lax.axis_index("c")
    b = wid // _WPB
    j0 = (wid % _WPB) * _JPW
    f0 = j0 * _NS_TOT

    for d in range(3):
        pltpu.sync_copy(xyzt.at[pl.ds((b * 3 + d) * _N, _N)],
                        xyz_v.at[pl.ds(d * _N, _N)])
        pltpu.sync_copy(newt.at[pl.ds((b * 3 + d) * _NPOINT + j0, _JPW)],
                        new_v.at[pl.ds(d * _JPW, _JPW)])
    pltpu.sync_copy(fps.at[pl.ds(b * _NPOINT + j0, _JPW)], fps_v)

    lane = _lane_iota()
    zeros = jnp.zeros((_L,), _i32)
    ones = jnp.ones((_L,), _i32)
    r2 = jnp.float32(_RADIUS * _RADIUS)

    def per_centroid(j, carry):
        jv = jnp.full((_L,), j, _i32)
        cx = plsc.load_gather(new_v, [jv])
        cy = plsc.load_gather(new_v, [jv + _JPW])
        cz = plsc.load_gather(new_v, [jv + 2 * _JPW])

        def cond(state):
            n, cntv = state
            return jnp.logical_and(jnp.max(cntv) < _NSAMPLE, n < _N)

        def body(state):
            n, cntv = state

            def chunk(rel, off):
                base = n + rel
                xs = xyz_v[pl.ds(base, _L)]
                ys = xyz_v[pl.ds(_N + base, _L)]
                zs = xyz_v[pl.ds(2 * _N + base, _L)]
                dx = xs - cx
                dy = ys - cy
                dz = zs - cz
                d2 = (dx * dx + dy * dy) + dz * dz
                mask = d2 < r2
                pos = off + plsc.cumsum(ones, mask=mask) - 1
                plsc.store_scatter(buf, [pos], lane + base, mask=mask)
                return off + plsc.all_reduce_population_count(mask)

            off = plsc.parallel_loop(0, _BLK, step=_L, unroll=_BLK // _L,
                                     carry=cntv)(chunk)
            return n + _BLK, off

        _, cntv = lax.while_loop(
            cond, body, (jnp.int32(0), jnp.zeros((_L,), _i32)))

        firstv = plsc.load_gather(buf, [zeros])
        firstv = jnp.where(cntv > 0, firstv, zeros)
        fpsv = plsc.load_gather(fps_v, [jv])
        base = j * _NS_TOT
        basev = jnp.full((_L,), base, _i32)
        plsc.store_scatter(idx_v, [basev], fpsv, mask=lane == 0)
        for g in range(2):
            bv = buf[pl.ds(g * _L, _L)]
            vals = jnp.where(lane + g * _L < cntv, bv, firstv)
            plsc.store_scatter(idx_v, [basev + 1 + g * _L + lane], vals)
        return carry

    lax.fori_loop(0, _JPW, per_centroid, jnp.int32(0))

    def feat_src(ch):
        return feat.at[pl.ds((b * _C + ch - 6) * _N, _N)]

    def out_dst(ch):
        return out.at[b, ch, pl.ds(j0, _JPW), :]

    def gather_loop(ob, tbl_off, center, src_ref):
        def chunk(k):
            iv = idx_v[pl.ds(k, _L)]
            if tbl_off:
                iv = iv + tbl_off
            vals = plsc.load_gather(src_ref, [iv])
            flat = jnp.full((_L,), k, _i32) + lane
            jv = lax.div(flat, jnp.full((_L,), _NS_TOT, _i32))
            if center:
                vals = vals - plsc.load_gather(new_v, [jv + tbl_off // _N * _JPW])
            plsc.store_scatter(ob, [jv, flat - jv * _NS_TOT], vals)

        plsc.parallel_loop(0, _FL2, step=_L, unroll=_GU)(chunk)

    pltpu.async_copy(feat_src(6), tbl0, si0)
    pltpu.async_copy(feat_src(7), tbl1, si1)

    for d in range(3):
        gather_loop(ob0, d * _N, True, xyz_v)
        pltpu.sync_copy(ob0, out_dst(d))
        pltpu.sync_copy(ob0, out_dst(d + 3))

    def pair(t, carry):
        ch0 = 6 + 2 * t
        for (tbl, si, ob, so, ch) in ((tbl0, si0, ob0, so0, ch0),
                                      (tbl1, si1, ob1, so1, ch0 + 1)):
            pltpu.make_async_copy(feat_src(ch), tbl, si).wait()

            @pl.when(t > 0)
            def _():
                pltpu.make_async_copy(ob, out_dst(ch - 2), so).wait()

            gather_loop(ob, 0, False, tbl)

            @pl.when(t < (_C // 2 - 1))
            def _():
                pltpu.async_copy(feat_src(ch + 2), tbl, si)

            pltpu.async_copy(ob, out_dst(ch), so)
        return carry

    lax.fori_loop(0, _C // 2, pair, jnp.int32(0))
    pltpu.make_async_copy(ob0, out_dst(_NCH - 2), so0).wait()
    pltpu.make_async_copy(ob1, out_dst(_NCH - 1), so1).wait()


@jax.jit
def kernel(xyz, new_xyz, features, fps_idx):
    xyzt = jnp.transpose(xyz, (0, 2, 1)).reshape(-1)
    newt = jnp.transpose(new_xyz, (0, 2, 1)).reshape(-1)
    feat = features.reshape(-1)
    fps = fps_idx.reshape(-1)
    mesh = plsc.VectorSubcoreMesh(core_axis_name="c", subcore_axis_name="s")

    fused = pl.kernel(
        _fused_body,
        out_type=jax.ShapeDtypeStruct((_B, _NCH, _NPOINT, _NS_TOT),
                                      jnp.float32),
        mesh=mesh,
        compiler_params=pltpu.CompilerParams(needs_layout_passes=False),
        scratch_types=[
            pltpu.VMEM((3 * _N,), jnp.float32),
            pltpu.VMEM((3 * _JPW,), jnp.float32),
            pltpu.VMEM((_JPW,), _i32),
            pltpu.VMEM((_FL2,), _i32),
            pltpu.VMEM((_NSAMPLE + _BLK,), _i32),
            pltpu.VMEM((_N,), jnp.float32),
            pltpu.VMEM((_N,), jnp.float32),
            pltpu.VMEM((_JPW, _NS_TOT), jnp.float32),
            pltpu.VMEM((_JPW, _NS_TOT), jnp.float32),
            pltpu.SemaphoreType.DMA,
            pltpu.SemaphoreType.DMA,
            pltpu.SemaphoreType.DMA,
            pltpu.SemaphoreType.DMA,
        ],
    )
    return fused(xyzt, newt, fps, feat)
